# baseline (speedup 1.0000x reference)
import functools

import jax
import jax.numpy as jnp
from jax import lax
from jax.experimental import pallas as pl
from jax.experimental.pallas import tpu as pltpu
from jax.experimental.pallas import tpu_sc as plsc

B, L, D = 16384, 200, 64
LP = 256
H, C = 50, 10
NC, NS = 2, 16
NW = NC * NS
RPW = B // NW
GROUP = 64
NGROUPS = RPW // GROUP
NBUF = 2
NLANE = 16
DV = D // NLANE


def _sc_body(idx_hbm, table_hbm, out_hbm, idx_v, rows_v, out_v, sem0, sem1):
    wid = lax.axis_index("s") * NC + lax.axis_index("c")
    sems = (sem0, sem1)

    def fire(buf, row):
        pltpu.make_async_copy(
            table_hbm.at[idx_v.at[row, pl.ds(0, L)]], rows_v.at[buf],
            sems[buf]
        ).start()

    def drain(buf):
        pltpu.make_async_copy(
            table_hbm.at[idx_v.at[0, pl.ds(0, L)]], rows_v.at[buf],
            sems[buf]
        ).wait()

    def accum_store(buf, out_row):
        rbuf = rows_v.at[buf]

        def it(i, acc):
            return tuple(acc[k] + rbuf[i, pl.ds(NLANE * k, NLANE)]
                         for k in range(DV))

        acc0 = tuple(jnp.zeros((NLANE,), jnp.float32) for _ in range(DV))
        acc = lax.fori_loop(0, L, it, acc0, unroll=8)
        for k in range(DV):
            out_v[out_row, pl.ds(NLANE * k, NLANE)] = acc[k]

    @pl.loop(0, NGROUPS)
    def _(g):
        pltpu.sync_copy(idx_hbm.at[pl.ds(wid * RPW + g * GROUP, GROUP), :],
                        idx_v)
        for b in range(NBUF):
            fire(b, b)

        @pl.loop(0, GROUP, step=NBUF)
        def _(r0):
            for b in range(NBUF):
                r = r0 + b
                drain(b)
                accum_store(b, g * GROUP + r)
                nxt = r + NBUF

                @pl.when(nxt < GROUP)
                def _():
                    fire(b, nxt)

    pltpu.sync_copy(out_v, out_hbm.at[pl.ds(wid * RPW, RPW), :])


_sc_pool = functools.partial(
    pl.kernel,
    out_type=jax.ShapeDtypeStruct((B, D), jnp.float32),
    mesh=plsc.VectorSubcoreMesh(core_axis_name="c", subcore_axis_name="s",
                                num_cores=NC, num_subcores=NS),
    scratch_types=[
        pltpu.VMEM((GROUP, LP), jnp.int32),
        pltpu.VMEM((NBUF, L, D), jnp.float32),
        pltpu.VMEM((RPW, D), jnp.float32),
        pltpu.SemaphoreType.DMA,
        pltpu.SemaphoreType.DMA,
    ],
    compiler_params=pltpu.CompilerParams(use_tc_tiling_on_sc=False),
)(_sc_body)


BM = 2048


def _mlp_body(x_ref, tl_ref, w1_ref, b1_ref, w2_ref, b2_ref, o_ref):
    x = x_ref[...] / tl_ref[...]
    h = jnp.maximum(
        jnp.dot(x, w1_ref[...], preferred_element_type=jnp.float32)
        + b1_ref[...], 0.0)
    o_ref[...] = (jnp.dot(h, w2_ref[...], preferred_element_type=jnp.float32)
                  + b2_ref[...])


def _mlp(pooled, text_len, W1, b1, W2, b2):
    return pl.pallas_call(
        _mlp_body,
        grid=(B // BM,),
        in_specs=[
            pl.BlockSpec((BM, D), lambda i: (i, 0)),
            pl.BlockSpec((BM, 1), lambda i: (i, 0)),
            pl.BlockSpec((D, H), lambda i: (0, 0)),
            pl.BlockSpec((1, H), lambda i: (0, 0)),
            pl.BlockSpec((H, C), lambda i: (0, 0)),
            pl.BlockSpec((1, C), lambda i: (0, 0)),
        ],
        out_specs=pl.BlockSpec((BM, C), lambda i: (i, 0)),
        out_shape=jax.ShapeDtypeStruct((B, C), jnp.float32),
    )(pooled, text_len.reshape(B, 1), W1, b1.reshape(1, H), W2,
      b2.reshape(1, C))


def kernel(input_text, text_len, emb_table, W1, b1, W2, b2):
    idx = jnp.pad(input_text.astype(jnp.int32), ((0, 0), (0, LP - L)))
    pooled = _sc_pool(idx, emb_table)
    return _mlp(pooled, text_len, W1, b1, W2, b2)


# speedup vs baseline: 1.1404x; 1.1404x over previous
import functools

import jax
import jax.numpy as jnp
from jax import lax
from jax.experimental import pallas as pl
from jax.experimental.pallas import tpu as pltpu
from jax.experimental.pallas import tpu_sc as plsc

B, L, D = 16384, 200, 64
VOCAB = 1000000
BKT = 2048
NBLK = -(-VOCAB // BKT)
PHALF = BKT // 2
VT = NBLK * BKT
LP = 256
H, C = 50, 10
NC, NS = 2, 16
NW = NC * NS
RPW = B // NW
GROUP = 64
NGROUPS = RPW // GROUP
NBUF = 2
NLANE = 16
DV = D // NLANE


def _sc_body(idx_hbm, table_hbm, out_hbm, idx_v, rows_v, out_v, sem0, sem1):
    wid = lax.axis_index("s") * NC + lax.axis_index("c")
    sems = (sem0, sem1)

    def fire(buf, row):
        pltpu.make_async_copy(
            table_hbm.at[idx_v.at[row, pl.ds(0, L)]], rows_v.at[buf],
            sems[buf]
        ).start()

    def drain(buf):
        pltpu.make_async_copy(
            table_hbm.at[idx_v.at[0, pl.ds(0, L)]], rows_v.at[buf],
            sems[buf]
        ).wait()

    def accum_store(buf, out_row):
        rbuf = rows_v.at[buf]

        def it(i, acc):
            return tuple(acc[k] + rbuf[i, pl.ds(NLANE * k, NLANE)]
                         for k in range(DV))

        acc0 = tuple(jnp.zeros((NLANE,), jnp.float32) for _ in range(DV))
        acc = lax.fori_loop(0, L, it, acc0, unroll=8)
        for k in range(DV):
            out_v[out_row, pl.ds(NLANE * k, NLANE)] = acc[k]

    @pl.loop(0, NGROUPS)
    def _(g):
        pltpu.sync_copy(idx_hbm.at[pl.ds(wid * RPW + g * GROUP, GROUP), :],
                        idx_v)

        @pl.loop(0, GROUP)
        def _(r):
            for c in range((L + NLANE - 1) // NLANE):
                v = idx_v[r, pl.ds(NLANE * c, NLANE)]
                rr = jnp.bitwise_and(v, BKT - 1)
                q = jnp.bitwise_and(rr, PHALF - 1)
                h = jax.lax.shift_right_logical(rr, 10)
                idx_v[r, pl.ds(NLANE * c, NLANE)] = (
                    (v - rr) + q + q + h)

        for b in range(NBUF):
            fire(b, b)

        @pl.loop(0, GROUP, step=NBUF)
        def _(r0):
            for b in range(NBUF):
                r = r0 + b
                drain(b)
                accum_store(b, g * GROUP + r)
                nxt = r + NBUF

                @pl.when(nxt < GROUP)
                def _():
                    fire(b, nxt)

    pltpu.sync_copy(out_v, out_hbm.at[pl.ds(wid * RPW, RPW), :])


_sc_pool = functools.partial(
    pl.kernel,
    out_type=jax.ShapeDtypeStruct((B, D), jnp.float32),
    mesh=plsc.VectorSubcoreMesh(core_axis_name="c", subcore_axis_name="s",
                                num_cores=NC, num_subcores=NS),
    scratch_types=[
        pltpu.VMEM((GROUP, LP), jnp.int32),
        pltpu.VMEM((NBUF, L, D), jnp.float32),
        pltpu.VMEM((RPW, D), jnp.float32),
        pltpu.SemaphoreType.DMA,
        pltpu.SemaphoreType.DMA,
    ],
    compiler_params=pltpu.CompilerParams(use_tc_tiling_on_sc=False),
)(_sc_body)


def _linearize_body(x_ref, o_ref):
    y = jnp.transpose(x_ref[...], (1, 0))
    o_ref[...] = jnp.concatenate([y[:PHALF], y[PHALF:]], axis=1)


def _linearize(table_t):
    return pl.pallas_call(
        _linearize_body,
        grid=(NBLK,),
        in_specs=[pl.BlockSpec((D, BKT), lambda i: (0, i))],
        out_specs=pl.BlockSpec((PHALF, 2 * D), lambda i: (i, 0)),
        out_shape=jax.ShapeDtypeStruct((NBLK * PHALF, 2 * D), jnp.float32),
    )(table_t)


BM = 2048


def _mlp_body(x_ref, tl_ref, w1_ref, b1_ref, w2_ref, b2_ref, o_ref):
    x = x_ref[...] / tl_ref[...]
    h = jnp.maximum(
        jnp.dot(x, w1_ref[...], preferred_element_type=jnp.float32)
        + b1_ref[...], 0.0)
    o_ref[...] = (jnp.dot(h, w2_ref[...], preferred_element_type=jnp.float32)
                  + b2_ref[...])


def _mlp(pooled, text_len, W1, b1, W2, b2):
    return pl.pallas_call(
        _mlp_body,
        grid=(B // BM,),
        in_specs=[
            pl.BlockSpec((BM, D), lambda i: (i, 0)),
            pl.BlockSpec((BM, 1), lambda i: (i, 0)),
            pl.BlockSpec((D, H), lambda i: (0, 0)),
            pl.BlockSpec((1, H), lambda i: (0, 0)),
            pl.BlockSpec((H, C), lambda i: (0, 0)),
            pl.BlockSpec((1, C), lambda i: (0, 0)),
        ],
        out_specs=pl.BlockSpec((BM, C), lambda i: (i, 0)),
        out_shape=jax.ShapeDtypeStruct((B, C), jnp.float32),
    )(pooled, text_len.reshape(B, 1), W1, b1.reshape(1, H), W2,
      b2.reshape(1, C))


def kernel(input_text, text_len, emb_table, W1, b1, W2, b2):
    idx = jnp.pad(input_text.astype(jnp.int32), ((0, 0), (0, LP - L)))
    table_lin = _linearize(emb_table.T).reshape(VT, D)
    pooled = _sc_pool(idx, table_lin)
    return _mlp(pooled, text_len, W1, b1, W2, b2)


# speedup vs baseline: 1.4192x; 1.2445x over previous
import functools

import jax
import jax.numpy as jnp
from jax import lax
from jax.experimental import pallas as pl
from jax.experimental.pallas import tpu as pltpu
from jax.experimental.pallas import tpu_sc as plsc

B, L, D = 16384, 200, 64
VOCAB = 1000000
BKT = 8192
NBLK = -(-VOCAB // BKT)
PHALF = BKT // 2
VT = NBLK * BKT
LP = 256
H, C = 50, 10
NC, NS = 2, 16
NW = NC * NS
RPW = B // NW
GROUP = 64
NGROUPS = RPW // GROUP
NBUF = 2
NLANE = 16
DV = D // NLANE


def _sc_body(idx_hbm, table_hbm, out_hbm, idx_v, rows_v, out_v, sem0, sem1):
    wid = lax.axis_index("s") * NC + lax.axis_index("c")
    sems = (sem0, sem1)

    def fire(buf, row):
        pltpu.make_async_copy(
            table_hbm.at[idx_v.at[row, pl.ds(0, L)]], rows_v.at[buf],
            sems[buf]
        ).start()

    def drain(buf):
        pltpu.make_async_copy(
            table_hbm.at[idx_v.at[0, pl.ds(0, L)]], rows_v.at[buf],
            sems[buf]
        ).wait()

    def accum_store(buf, out_row):
        rbuf = rows_v.at[buf]

        def it(i, acc):
            return tuple(acc[k] + rbuf[i, pl.ds(NLANE * k, NLANE)]
                         for k in range(DV))

        acc0 = tuple(jnp.zeros((NLANE,), jnp.float32) for _ in range(DV))
        acc = lax.fori_loop(0, L, it, acc0, unroll=8)
        for k in range(DV):
            out_v[out_row, pl.ds(NLANE * k, NLANE)] = acc[k]

    @pl.loop(0, NGROUPS)
    def _(g):
        pltpu.sync_copy(idx_hbm.at[pl.ds(wid * RPW + g * GROUP, GROUP), :],
                        idx_v)

        @pl.loop(0, GROUP)
        def _(r):
            for c in range((L + NLANE - 1) // NLANE):
                v = idx_v[r, pl.ds(NLANE * c, NLANE)]
                rr = jnp.bitwise_and(v, BKT - 1)
                q = jnp.bitwise_and(rr, PHALF - 1)
                h = jax.lax.shift_right_logical(rr, PHALF.bit_length() - 1)
                idx_v[r, pl.ds(NLANE * c, NLANE)] = (
                    (v - rr) + q + q + h)

        for b in range(NBUF):
            fire(b, b)

        @pl.loop(0, GROUP, step=NBUF)
        def _(r0):
            for b in range(NBUF):
                r = r0 + b
                drain(b)
                accum_store(b, g * GROUP + r)
                nxt = r + NBUF

                @pl.when(nxt < GROUP)
                def _():
                    fire(b, nxt)

    pltpu.sync_copy(out_v, out_hbm.at[pl.ds(wid * RPW, RPW), :])


_sc_pool = functools.partial(
    pl.kernel,
    out_type=jax.ShapeDtypeStruct((B, D), jnp.float32),
    mesh=plsc.VectorSubcoreMesh(core_axis_name="c", subcore_axis_name="s",
                                num_cores=NC, num_subcores=NS),
    scratch_types=[
        pltpu.VMEM((GROUP, LP), jnp.int32),
        pltpu.VMEM((NBUF, L, D), jnp.float32),
        pltpu.VMEM((RPW, D), jnp.float32),
        pltpu.SemaphoreType.DMA,
        pltpu.SemaphoreType.DMA,
    ],
    compiler_params=pltpu.CompilerParams(use_tc_tiling_on_sc=False),
)(_sc_body)


def _linearize_body(x_ref, o_ref):
    y = jnp.transpose(x_ref[...], (1, 0))
    o_ref[...] = jnp.concatenate([y[:PHALF], y[PHALF:]], axis=1)


def _linearize(table_t):
    return pl.pallas_call(
        _linearize_body,
        grid=(NBLK,),
        in_specs=[pl.BlockSpec((D, BKT), lambda i: (0, i))],
        out_specs=pl.BlockSpec((PHALF, 2 * D), lambda i: (i, 0)),
        out_shape=jax.ShapeDtypeStruct((NBLK * PHALF, 2 * D), jnp.float32),
    )(table_t)


BM = 2048


def _mlp_body(x_ref, tl_ref, w1_ref, b1_ref, w2_ref, b2_ref, o_ref):
    x = x_ref[...] / tl_ref[...]
    h = jnp.maximum(
        jnp.dot(x, w1_ref[...], preferred_element_type=jnp.float32)
        + b1_ref[...], 0.0)
    o_ref[...] = (jnp.dot(h, w2_ref[...], preferred_element_type=jnp.float32)
                  + b2_ref[...])


def _mlp(pooled, text_len, W1, b1, W2, b2):
    return pl.pallas_call(
        _mlp_body,
        grid=(B // BM,),
        in_specs=[
            pl.BlockSpec((BM, D), lambda i: (i, 0)),
            pl.BlockSpec((BM, 1), lambda i: (i, 0)),
            pl.BlockSpec((D, H), lambda i: (0, 0)),
            pl.BlockSpec((1, H), lambda i: (0, 0)),
            pl.BlockSpec((H, C), lambda i: (0, 0)),
            pl.BlockSpec((1, C), lambda i: (0, 0)),
        ],
        out_specs=pl.BlockSpec((BM, C), lambda i: (i, 0)),
        out_shape=jax.ShapeDtypeStruct((B, C), jnp.float32),
    )(pooled, text_len.reshape(B, 1), W1, b1.reshape(1, H), W2,
      b2.reshape(1, C))


def kernel(input_text, text_len, emb_table, W1, b1, W2, b2):
    idx = jnp.pad(input_text.astype(jnp.int32), ((0, 0), (0, LP - L)))
    table_lin = _linearize(emb_table.T).reshape(VT, D)
    pooled = _sc_pool(idx, table_lin)
    return _mlp(pooled, text_len, W1, b1, W2, b2)


# speedup vs baseline: 1.6679x; 1.1752x over previous
import functools

import jax
import jax.numpy as jnp
from jax import lax
from jax.experimental import pallas as pl
from jax.experimental.pallas import tpu as pltpu
from jax.experimental.pallas import tpu_sc as plsc

B, L, D = 16384, 200, 64
VOCAB = 1000000
BKT = 8192
NBLK = -(-VOCAB // BKT)
QB = BKT // 4
VT = NBLK * BKT
D2 = D // 2
LP = 256
H, C = 50, 10
NC, NS = 2, 16
NW = NC * NS
RPW = B // NW
GROUP = 64
NGROUPS = RPW // GROUP
NBUF = 2
NLANE = 16
DV = D // NLANE


def _sc_body(idx_hbm, table_hbm, out_hbm, idx_v, rows_v, out_v, sem0, sem1):
    wid = lax.axis_index("s") * NC + lax.axis_index("c")
    sems = (sem0, sem1)

    def fire(buf, row):
        pltpu.make_async_copy(
            table_hbm.at[idx_v.at[row, pl.ds(0, L)]], rows_v.at[buf],
            sems[buf]
        ).start()

    def drain(buf):
        pltpu.make_async_copy(
            table_hbm.at[idx_v.at[0, pl.ds(0, L)]], rows_v.at[buf],
            sems[buf]
        ).wait()

    def accum_store(buf, out_row):
        rbuf = rows_v.at[buf]
        mask_hi = jnp.full((NLANE,), -65536, jnp.int32)

        def it(i, acc):
            out = list(acc)
            for k in range(2):
                w = jax.lax.bitcast_convert_type(
                    rbuf[i, pl.ds(NLANE * k, NLANE)], jnp.int32)
                lo = jax.lax.bitcast_convert_type(
                    jnp.left_shift(w, 16), jnp.float32)
                hi = jax.lax.bitcast_convert_type(
                    jnp.bitwise_and(w, mask_hi), jnp.float32)
                out[k] = out[k] + lo
                out[2 + k] = out[2 + k] + hi
            return tuple(out)

        acc0 = tuple(jnp.zeros((NLANE,), jnp.float32) for _ in range(DV))
        acc = lax.fori_loop(0, L, it, acc0, unroll=8)
        for k in range(DV):
            out_v[out_row, pl.ds(NLANE * k, NLANE)] = acc[k]

    @pl.loop(0, NGROUPS)
    def _(g):
        pltpu.sync_copy(idx_hbm.at[pl.ds(wid * RPW + g * GROUP, GROUP), :],
                        idx_v)

        @pl.loop(0, GROUP)
        def _(r):
            for c in range((L + NLANE - 1) // NLANE):
                v = idx_v[r, pl.ds(NLANE * c, NLANE)]
                rr = jnp.bitwise_and(v, BKT - 1)
                q = jnp.bitwise_and(rr, QB - 1)
                h = jax.lax.shift_right_logical(rr, QB.bit_length() - 1)
                idx_v[r, pl.ds(NLANE * c, NLANE)] = (
                    (v - rr) + jnp.left_shift(q, 2) + h)

        for b in range(NBUF):
            fire(b, b)

        @pl.loop(0, GROUP, step=NBUF)
        def _(r0):
            for b in range(NBUF):
                r = r0 + b
                drain(b)
                accum_store(b, g * GROUP + r)
                nxt = r + NBUF

                @pl.when(nxt < GROUP)
                def _():
                    fire(b, nxt)

    pltpu.sync_copy(out_v, out_hbm.at[pl.ds(wid * RPW, RPW), :])


@functools.cache
def _sc_pool_fn():
    return functools.partial(
        pl.kernel,
        out_type=jax.ShapeDtypeStruct((B, D), jnp.float32),
        mesh=plsc.VectorSubcoreMesh(core_axis_name="c", subcore_axis_name="s",
                                    num_cores=NC, num_subcores=NS),
        scratch_types=[
            pltpu.VMEM((GROUP, LP), jnp.int32),
            pltpu.VMEM((NBUF, L, D2), jnp.float32),
            pltpu.VMEM((RPW, D), jnp.float32),
            pltpu.SemaphoreType.DMA,
            pltpu.SemaphoreType.DMA,
        ],
        compiler_params=pltpu.CompilerParams(use_tc_tiling_on_sc=False),
    )(_sc_body)


def _linearize_body(x_ref, o_ref):
    xi = jax.lax.bitcast_convert_type(x_ref[...], jnp.int32)
    half = jnp.full((1, 1), 0x8000, jnp.int32)
    mask_hi = jnp.full((1, 1), -65536, jnp.int32)
    lo = jax.lax.shift_right_logical(xi[:D2] + half, 16)
    hi = jnp.bitwise_and(xi[D2:] + half, mask_hi)
    w = jnp.transpose(jnp.bitwise_or(lo, hi), (1, 0))
    o_ref[...] = jax.lax.bitcast_convert_type(
        jnp.concatenate([w[0:QB], w[QB:2 * QB], w[2 * QB:3 * QB],
                         w[3 * QB:]], axis=1), jnp.float32)


def _linearize(table_t):
    return pl.pallas_call(
        _linearize_body,
        grid=(NBLK,),
        in_specs=[pl.BlockSpec((D, BKT), lambda i: (0, i))],
        out_specs=pl.BlockSpec((QB, 2 * D), lambda i: (i, 0)),
        out_shape=jax.ShapeDtypeStruct((NBLK * QB, 2 * D), jnp.float32),
    )(table_t)


BM = 2048


def _mlp_body(x_ref, tl_ref, w1_ref, b1_ref, w2_ref, b2_ref, o_ref):
    x = x_ref[...] / tl_ref[...]
    h = jnp.maximum(
        jnp.dot(x, w1_ref[...], preferred_element_type=jnp.float32)
        + b1_ref[...], 0.0)
    o_ref[...] = (jnp.dot(h, w2_ref[...], preferred_element_type=jnp.float32)
                  + b2_ref[...])


def _mlp(pooled, text_len, W1, b1, W2, b2):
    return pl.pallas_call(
        _mlp_body,
        grid=(B // BM,),
        in_specs=[
            pl.BlockSpec((BM, D), lambda i: (i, 0)),
            pl.BlockSpec((BM, 1), lambda i: (i, 0)),
            pl.BlockSpec((D, H), lambda i: (0, 0)),
            pl.BlockSpec((1, H), lambda i: (0, 0)),
            pl.BlockSpec((H, C), lambda i: (0, 0)),
            pl.BlockSpec((1, C), lambda i: (0, 0)),
        ],
        out_specs=pl.BlockSpec((BM, C), lambda i: (i, 0)),
        out_shape=jax.ShapeDtypeStruct((B, C), jnp.float32),
    )(pooled, text_len.reshape(B, 1), W1, b1.reshape(1, H), W2,
      b2.reshape(1, C))


def kernel(input_text, text_len, emb_table, W1, b1, W2, b2):
    idx = jnp.pad(input_text.astype(jnp.int32), ((0, 0), (0, LP - L)))
    table_lin = _linearize(emb_table.T).reshape(VT, D2)
    pooled = _sc_pool_fn()(idx, table_lin)
    return _mlp(pooled, text_len, W1, b1, W2, b2)


# speedup vs baseline: 1.7142x; 1.0278x over previous
import functools

import jax
import jax.numpy as jnp
from jax import lax
from jax.experimental import pallas as pl
from jax.experimental.pallas import tpu as pltpu
from jax.experimental.pallas import tpu_sc as plsc

B, L, D = 16384, 200, 64
VOCAB = 1000000
BKT = 8192
NBLK = -(-VOCAB // BKT)
QB = BKT // 4
VT = NBLK * BKT
D2 = D // 2
LP = 256
H, C = 50, 10
NC, NS = 2, 16
NW = NC * NS
RPW = B // NW
GROUP = 64
NGROUPS = RPW // GROUP
NBUF = 2
NLANE = 16
DV = D // NLANE


def _sc_body(idx_hbm, table_hbm, out_hbm, idx_v, rows_v, out_v, sem0, sem1):
    wid = lax.axis_index("s") * NC + lax.axis_index("c")
    sems = (sem0, sem1)

    def fire(buf, row):
        pltpu.make_async_copy(
            table_hbm.at[idx_v.at[row, pl.ds(0, L)]], rows_v.at[buf],
            sems[buf]
        ).start()

    def drain(buf):
        pltpu.make_async_copy(
            table_hbm.at[idx_v.at[0, pl.ds(0, L)]], rows_v.at[buf],
            sems[buf]
        ).wait()

    def accum_store(buf, out_row):
        rbuf = rows_v.at[buf]

        def it(i, acc):
            out = list(acc)
            for k in range(2):
                wf = rbuf[i, pl.ds(NLANE * k, NLANE)]
                w = jax.lax.bitcast_convert_type(wf, jnp.int32)
                lo = jax.lax.bitcast_convert_type(
                    jnp.left_shift(w, 16), jnp.float32)
                out[k] = out[k] + lo
                out[2 + k] = out[2 + k] + wf
            return tuple(out)

        acc0 = tuple(jnp.zeros((NLANE,), jnp.float32) for _ in range(DV))
        acc = lax.fori_loop(0, L, it, acc0, unroll=8)
        for k in range(DV):
            out_v[out_row, pl.ds(NLANE * k, NLANE)] = acc[k]

    @pl.loop(0, NGROUPS)
    def _(g):
        pltpu.sync_copy(idx_hbm.at[pl.ds(wid * RPW + g * GROUP, GROUP), :],
                        idx_v)

        @pl.loop(0, GROUP)
        def _(r):
            for c in range((L + NLANE - 1) // NLANE):
                v = idx_v[r, pl.ds(NLANE * c, NLANE)]
                rr = jnp.bitwise_and(v, BKT - 1)
                q = jnp.bitwise_and(rr, QB - 1)
                h = jax.lax.shift_right_logical(rr, QB.bit_length() - 1)
                idx_v[r, pl.ds(NLANE * c, NLANE)] = (
                    (v - rr) + jnp.left_shift(q, 2) + h)

        for b in range(NBUF):
            fire(b, b)

        @pl.loop(0, GROUP, step=NBUF)
        def _(r0):
            for b in range(NBUF):
                r = r0 + b
                drain(b)
                accum_store(b, g * GROUP + r)
                nxt = r + NBUF

                @pl.when(nxt < GROUP)
                def _():
                    fire(b, nxt)

    pltpu.sync_copy(out_v, out_hbm.at[pl.ds(wid * RPW, RPW), :])


@functools.cache
def _sc_pool_fn():
    return functools.partial(
        pl.kernel,
        out_type=jax.ShapeDtypeStruct((B, D), jnp.float32),
        mesh=plsc.VectorSubcoreMesh(core_axis_name="c", subcore_axis_name="s",
                                    num_cores=NC, num_subcores=NS),
        scratch_types=[
            pltpu.VMEM((GROUP, LP), jnp.int32),
            pltpu.VMEM((NBUF, L, D2), jnp.float32),
            pltpu.VMEM((RPW, D), jnp.float32),
            pltpu.SemaphoreType.DMA,
            pltpu.SemaphoreType.DMA,
        ],
        compiler_params=pltpu.CompilerParams(use_tc_tiling_on_sc=False),
    )(_sc_body)


def _linearize_body(x_ref, o_ref):
    xi = jax.lax.bitcast_convert_type(x_ref[...], jnp.int32)
    half = jnp.full((1, 1), 0x8000, jnp.int32)
    mask_hi = jnp.full((1, 1), -65536, jnp.int32)
    lo = jax.lax.shift_right_logical(xi[:D2] + half, 16)
    hi = jnp.bitwise_and(xi[D2:] + half, mask_hi)
    w = jnp.transpose(jnp.bitwise_or(lo, hi), (1, 0))
    o_ref[...] = jax.lax.bitcast_convert_type(
        jnp.concatenate([w[0:QB], w[QB:2 * QB], w[2 * QB:3 * QB],
                         w[3 * QB:]], axis=1), jnp.float32)


def _linearize(table_t):
    return pl.pallas_call(
        _linearize_body,
        grid=(NBLK,),
        in_specs=[pl.BlockSpec((D, BKT), lambda i: (0, i))],
        out_specs=pl.BlockSpec((QB, 2 * D), lambda i: (i, 0)),
        out_shape=jax.ShapeDtypeStruct((NBLK * QB, 2 * D), jnp.float32),
    )(table_t)


BM = 2048


def _mlp_body(x_ref, tl_ref, w1_ref, b1_ref, w2_ref, b2_ref, o_ref):
    x = x_ref[...] / tl_ref[...]
    h = jnp.maximum(
        jnp.dot(x, w1_ref[...], preferred_element_type=jnp.float32)
        + b1_ref[...], 0.0)
    o_ref[...] = (jnp.dot(h, w2_ref[...], preferred_element_type=jnp.float32)
                  + b2_ref[...])


def _mlp(pooled, text_len, W1, b1, W2, b2):
    return pl.pallas_call(
        _mlp_body,
        grid=(B // BM,),
        in_specs=[
            pl.BlockSpec((BM, D), lambda i: (i, 0)),
            pl.BlockSpec((BM, 1), lambda i: (i, 0)),
            pl.BlockSpec((D, H), lambda i: (0, 0)),
            pl.BlockSpec((1, H), lambda i: (0, 0)),
            pl.BlockSpec((H, C), lambda i: (0, 0)),
            pl.BlockSpec((1, C), lambda i: (0, 0)),
        ],
        out_specs=pl.BlockSpec((BM, C), lambda i: (i, 0)),
        out_shape=jax.ShapeDtypeStruct((B, C), jnp.float32),
    )(pooled, text_len.reshape(B, 1), W1, b1.reshape(1, H), W2,
      b2.reshape(1, C))


def kernel(input_text, text_len, emb_table, W1, b1, W2, b2):
    idx = jnp.pad(input_text.astype(jnp.int32), ((0, 0), (0, LP - L)))
    table_lin = _linearize(emb_table.T).reshape(VT, D2)
    pooled = _sc_pool_fn()(idx, table_lin)
    return _mlp(pooled, text_len, W1, b1, W2, b2)


# speedup vs baseline: 2.1336x; 1.2447x over previous
import functools

import jax
import jax.numpy as jnp
from jax import lax
from jax.experimental import pallas as pl
from jax.experimental.pallas import tpu as pltpu
from jax.experimental.pallas import tpu_sc as plsc

B, L, D = 16384, 200, 64
VOCAB = 1000000
BKT = 8192
NBLK = -(-VOCAB // BKT)
QB = BKT // 4
VT = NBLK * BKT
D2 = D // 2
LP = 256
H, C = 50, 10
NC, NS = 2, 16
NW = NC * NS
RPW = B // NW
GROUP = 64
NGROUPS = RPW // GROUP
NBUF = 4
NLANE = 16
DV = D // NLANE


def _sc_body(idx_hbm, table_hbm, out_hbm, idx_v, rows_v, out_v,
             sem0, sem1, sem2, sem3):
    wid = lax.axis_index("s") * NC + lax.axis_index("c")
    sems = (sem0, sem1, sem2, sem3)

    def fire(buf, row):
        pltpu.make_async_copy(
            table_hbm.at[idx_v.at[row, pl.ds(0, L)]], rows_v.at[buf],
            sems[buf]
        ).start()

    def drain(buf):
        pltpu.make_async_copy(
            table_hbm.at[idx_v.at[0, pl.ds(0, L)]], rows_v.at[buf],
            sems[buf]
        ).wait()

    def accum_store(buf, out_row):
        rbuf = rows_v.at[buf]

        def it(i, acc):
            out = list(acc)
            for k in range(2):
                wf = rbuf[i, pl.ds(NLANE * k, NLANE)]
                w = jax.lax.bitcast_convert_type(wf, jnp.int32)
                lo = jax.lax.bitcast_convert_type(
                    jnp.left_shift(w, 16), jnp.float32)
                out[k] = out[k] + lo
                out[2 + k] = out[2 + k] + wf
            return tuple(out)

        acc0 = tuple(jnp.zeros((NLANE,), jnp.float32) for _ in range(DV))
        acc = lax.fori_loop(0, L, it, acc0, unroll=10)
        for k in range(DV):
            out_v[out_row, pl.ds(NLANE * k, NLANE)] = acc[k]

    @pl.loop(0, NGROUPS)
    def _(g):
        pltpu.sync_copy(idx_hbm.at[pl.ds(wid * RPW + g * GROUP, GROUP), :],
                        idx_v)

        @pl.loop(0, GROUP)
        def _(r):
            for c in range((L + NLANE - 1) // NLANE):
                v = idx_v[r, pl.ds(NLANE * c, NLANE)]
                rr = jnp.bitwise_and(v, BKT - 1)
                q = jnp.bitwise_and(rr, QB - 1)
                h = jax.lax.shift_right_logical(rr, QB.bit_length() - 1)
                idx_v[r, pl.ds(NLANE * c, NLANE)] = (
                    (v - rr) + jnp.left_shift(q, 2) + h)

        for b in range(NBUF):
            fire(b, b)

        @pl.loop(0, GROUP, step=NBUF)
        def _(r0):
            for b in range(NBUF):
                r = r0 + b
                drain(b)
                accum_store(b, g * GROUP + r)
                nxt = r + NBUF

                @pl.when(nxt < GROUP)
                def _():
                    fire(b, nxt)

    pltpu.sync_copy(out_v, out_hbm.at[pl.ds(wid * RPW, RPW), :])


@functools.cache
def _sc_pool_fn():
    return functools.partial(
        pl.kernel,
        out_type=jax.ShapeDtypeStruct((B, D), jnp.float32),
        mesh=plsc.VectorSubcoreMesh(core_axis_name="c", subcore_axis_name="s",
                                    num_cores=NC, num_subcores=NS),
        scratch_types=[
            pltpu.VMEM((GROUP, LP), jnp.int32),
            pltpu.VMEM((NBUF, L, D2), jnp.float32),
            pltpu.VMEM((RPW, D), jnp.float32),
            pltpu.SemaphoreType.DMA,
            pltpu.SemaphoreType.DMA,
            pltpu.SemaphoreType.DMA,
            pltpu.SemaphoreType.DMA,
        ],
        compiler_params=pltpu.CompilerParams(use_tc_tiling_on_sc=False),
    )(_sc_body)


def _linearize_body(x_ref, o_ref):
    xi = jax.lax.bitcast_convert_type(x_ref[...], jnp.int32)
    half = jnp.full((1, 1), 0x8000, jnp.int32)
    mask_hi = jnp.full((1, 1), -65536, jnp.int32)
    lo = jax.lax.shift_right_logical(xi[:D2] + half, 16)
    hi = jnp.bitwise_and(xi[D2:] + half, mask_hi)
    w = jnp.transpose(jnp.bitwise_or(lo, hi), (1, 0))
    o_ref[...] = jax.lax.bitcast_convert_type(
        jnp.concatenate([w[0:QB], w[QB:2 * QB], w[2 * QB:3 * QB],
                         w[3 * QB:]], axis=1), jnp.float32)


def _linearize(table_t):
    return pl.pallas_call(
        _linearize_body,
        grid=(NBLK,),
        in_specs=[pl.BlockSpec((D, BKT), lambda i: (0, i))],
        out_specs=pl.BlockSpec((QB, 2 * D), lambda i: (i, 0)),
        out_shape=jax.ShapeDtypeStruct((NBLK * QB, 2 * D), jnp.float32),
    )(table_t)


BM = 2048


def _mlp_body(x_ref, tl_ref, w1_ref, b1_ref, w2_ref, b2_ref, o_ref):
    x = x_ref[...] / tl_ref[...]
    h = jnp.maximum(
        jnp.dot(x, w1_ref[...], preferred_element_type=jnp.float32)
        + b1_ref[...], 0.0)
    o_ref[...] = (jnp.dot(h, w2_ref[...], preferred_element_type=jnp.float32)
                  + b2_ref[...])


def _mlp(pooled, text_len, W1, b1, W2, b2):
    return pl.pallas_call(
        _mlp_body,
        grid=(B // BM,),
        in_specs=[
            pl.BlockSpec((BM, D), lambda i: (i, 0)),
            pl.BlockSpec((BM, 1), lambda i: (i, 0)),
            pl.BlockSpec((D, H), lambda i: (0, 0)),
            pl.BlockSpec((1, H), lambda i: (0, 0)),
            pl.BlockSpec((H, C), lambda i: (0, 0)),
            pl.BlockSpec((1, C), lambda i: (0, 0)),
        ],
        out_specs=pl.BlockSpec((BM, C), lambda i: (i, 0)),
        out_shape=jax.ShapeDtypeStruct((B, C), jnp.float32),
    )(pooled, text_len.reshape(B, 1), W1, b1.reshape(1, H), W2,
      b2.reshape(1, C))


def kernel(input_text, text_len, emb_table, W1, b1, W2, b2):
    idx = jnp.pad(input_text.astype(jnp.int32), ((0, 0), (0, LP - L)))
    table_lin = _linearize(emb_table.T).reshape(VT, D2)
    pooled = _sc_pool_fn()(idx, table_lin)
    return _mlp(pooled, text_len, W1, b1, W2, b2)
